# baseline (speedup 1.0000x reference)
import jax
import jax.numpy as jnp
from jax import lax
from jax.experimental import pallas as pl
from jax.experimental.pallas import tpu as pltpu
from jax.experimental.pallas import tpu_sc as plsc

N_ENTITIES = 1000000
N_RELATIONS = 1000
D = 64
B = 16384
N_NEG = 4
NB = B * N_NEG

NC = 2
NS = 16
NW = NC * NS
L = 16

P = B // NW
NIDX = P // 128
NG = P // L


def _neg_sqrt(x):
    i = plsc.bitcast(x, jnp.int32)
    i = 0x5F3759DF - lax.shift_right_arithmetic(i, 1)
    y = plsc.bitcast(i, jnp.float32)
    half = x * (-0.5)
    for _ in range(3):
        y = y * (1.5 + half * y * y)
    return -(x * y)


def _scores_for_chunk(h_v, t_v, r_v, out_v):
    iota = lax.iota(jnp.int32, L)

    def group_body(g, _):
        rows = g * L + iota
        acc = jnp.zeros((L,), jnp.float32)
        for d in range(D):
            cols = jnp.full((L,), d, jnp.int32)
            hv = plsc.load_gather(h_v, [rows, cols])
            tv = plsc.load_gather(t_v, [rows, cols])
            rv = plsc.load_gather(r_v, [rows, cols])
            diff = hv + rv - tv
            acc = acc + diff * diff
        out_v[pl.ds(g * L, L)] = _neg_sqrt(acc + 1e-12)
        return ()

    lax.fori_loop(0, NG, group_body, (), unroll=False)


def _stage_idx(src_hbm, base, idx_v):
    for j in range(NIDX):
        pltpu.sync_copy(src_hbm.at[pl.ds(base + j * 128, 128)], idx_v.at[j])


def _gather_rows(table_hbm, idx_v, rows_v, sem):
    copies = [
        pltpu.async_copy(
            table_hbm.at[idx_v.at[j]], rows_v.at[pl.ds(j * 128, 128)], sem
        )
        for j in range(NIDX)
    ]
    return copies


def _sc_kernel(
    heads,
    tails,
    relations,
    negative_head,
    negative_tails,
    entity_emb,
    relation_emb,
    pos_out,
    neg_out,
    idx_a,
    idx_b,
    h_v,
    t_v,
    r_v,
    scores_v,
    sem,
):
    wid = lax.axis_index("s") * NC + lax.axis_index("c")
    base = wid * P

    _stage_idx(relations, base, idx_a)
    r_copies = _gather_rows(relation_emb, idx_a, r_v, sem)

    _stage_idx(heads, base, idx_a)
    _stage_idx(tails, base, idx_b)
    h_copies = _gather_rows(entity_emb, idx_a, h_v, sem)
    t_copies = _gather_rows(entity_emb, idx_b, t_v, sem)
    for c in r_copies + h_copies + t_copies:
        c.wait()
    _scores_for_chunk(h_v, t_v, r_v, scores_v)
    for k in range(N_NEG):
        pltpu.sync_copy(scores_v, pos_out.at[pl.ds(k * B + base, P)])

    for k in range(N_NEG):
        nbase = k * B + base
        _stage_idx(negative_head, nbase, idx_a)
        _stage_idx(negative_tails, nbase, idx_b)
        h_copies = _gather_rows(entity_emb, idx_a, h_v, sem)
        t_copies = _gather_rows(entity_emb, idx_b, t_v, sem)
        for c in h_copies + t_copies:
            c.wait()
        _scores_for_chunk(h_v, t_v, r_v, scores_v)
        pltpu.sync_copy(scores_v, neg_out.at[pl.ds(nbase, P)])


@jax.jit
def _run(heads, tails, relations, negative_head, negative_tails, entity_emb, relation_emb):
    mesh = plsc.VectorSubcoreMesh(
        core_axis_name="c", subcore_axis_name="s", num_cores=NC, num_subcores=NS
    )
    f = pl.kernel(
        _sc_kernel,
        out_type=(
            jax.ShapeDtypeStruct((NB,), jnp.float32),
            jax.ShapeDtypeStruct((NB,), jnp.float32),
        ),
        mesh=mesh,
        compiler_params=pltpu.CompilerParams(
            needs_layout_passes=False, use_tc_tiling_on_sc=False
        ),
        scratch_types=[
            pltpu.VMEM((NIDX, 128), jnp.int32),
            pltpu.VMEM((NIDX, 128), jnp.int32),
            pltpu.VMEM((P, D), jnp.float32),
            pltpu.VMEM((P, D), jnp.float32),
            pltpu.VMEM((P, D), jnp.float32),
            pltpu.VMEM((P,), jnp.float32),
            pltpu.SemaphoreType.DMA,
        ],
    )
    return f(heads, tails, relations, negative_head, negative_tails, entity_emb, relation_emb)


def kernel(heads, tails, relations, negative_head, negative_tails, entity_emb, relation_emb):
    return _run(
        heads.astype(jnp.int32),
        tails.astype(jnp.int32),
        relations.astype(jnp.int32),
        negative_head.astype(jnp.int32),
        negative_tails.astype(jnp.int32),
        entity_emb,
        relation_emb,
    )


# speedup vs baseline: 1.0962x; 1.0962x over previous
import jax
import jax.numpy as jnp
from jax import lax
from jax.experimental import pallas as pl
from jax.experimental.pallas import tpu as pltpu
from jax.experimental.pallas import tpu_sc as plsc

N_ENTITIES = 1000000
N_RELATIONS = 1000
D = 64
B = 16384
N_NEG = 4
NB = B * N_NEG

NC = 2
NS = 16
NW = NC * NS
L = 16

P = B // NW
C = 256
NSTAGE = (P + N_NEG * P) // C
PAD = 64
NSL = C // 128


def _neg_sqrt(x):
    i = plsc.bitcast(x, jnp.int32)
    i = 0x5F3759DF - lax.shift_right_arithmetic(i, 1)
    y = plsc.bitcast(i, jnp.float32)
    half = x * (-0.5)
    for _ in range(3):
        y = y * (1.5 + half * y * y)
    return -(x * y)


def _sc_kernel(
    heads,
    tails,
    relations,
    negative_head,
    negative_tails,
    entity_emb,
    relation_emb,
    pos_out,
    neg_out,
    idx_h,
    idx_t,
    idx_r,
    hbuf,
    tbuf,
    r_v,
    scores_v,
    sem_i,
    sem_r,
    sem_g,
    sem_o,
):
    wid = lax.axis_index("s") * NC + lax.axis_index("c")
    base = wid * P
    iota = lax.iota(jnp.int32, L)

    idx_copies = []
    for j in range(P // 128):
        idx_copies.append(
            pltpu.async_copy(
                relations.at[pl.ds(base + j * 128, 128)], idx_r.at[j], sem_i
            )
        )
        idx_copies.append(
            pltpu.async_copy(heads.at[pl.ds(base + j * 128, 128)], idx_h.at[j], sem_i)
        )
        idx_copies.append(
            pltpu.async_copy(tails.at[pl.ds(base + j * 128, 128)], idx_t.at[j], sem_i)
        )
    for k in range(N_NEG):
        for j in range(P // 128):
            src = k * B + base + j * 128
            row = P // 128 + k * (P // 128) + j
            idx_copies.append(
                pltpu.async_copy(
                    negative_head.at[pl.ds(src, 128)], idx_h.at[row], sem_i
                )
            )
            idx_copies.append(
                pltpu.async_copy(
                    negative_tails.at[pl.ds(src, 128)], idx_t.at[row], sem_i
                )
            )
    for c in idx_copies:
        c.wait()

    r_copies = [
        pltpu.async_copy(
            relation_emb.at[idx_r.at[j]],
            r_v.at[pl.ds(j * 128, 128)],
            sem_r,
        )
        for j in range(P // 128)
    ]
    for c in r_copies:
        c.wait()

    def fire(s):
        sp = s % 2
        for j in range(NSL):
            row = NSL * s + j
            pltpu.async_copy(
                entity_emb.at[idx_h.at[row]],
                hbuf.at[sp].at[pl.ds(j * 128, 128)],
                sem_g,
            )
            pltpu.async_copy(
                entity_emb.at[idx_t.at[row]],
                tbuf.at[sp].at[pl.ds(j * 128, 128)],
                sem_g,
            )

    def wait_stage(s):
        sp = s % 2
        for j in range(NSL):
            row = NSL * s + j
            pltpu.make_async_copy(
                entity_emb.at[idx_h.at[row]],
                hbuf.at[sp].at[pl.ds(j * 128, 128)],
                sem_g,
            ).wait()
            pltpu.make_async_copy(
                entity_emb.at[idx_t.at[row]],
                tbuf.at[sp].at[pl.ds(j * 128, 128)],
                sem_g,
            ).wait()

    fire(jnp.int32(0))

    def stage_body(s, _):
        wait_stage(s)

        @pl.when(s < NSTAGE - 1)
        def _():
            fire(s + 1)

        sp16 = jnp.full((L,), s % 2, jnp.int32)
        roff16 = jnp.full((L,), (s % 2) * C, jnp.int32)

        def group(g, _):
            rows = g * L + iota
            acc = jnp.zeros((L,), jnp.float32)
            for d in range(D):
                cols = jnp.full((L,), d, jnp.int32)
                hv = plsc.load_gather(hbuf, [sp16, rows, cols])
                tv = plsc.load_gather(tbuf, [sp16, rows, cols])
                rv = plsc.load_gather(r_v, [roff16 + rows, cols])
                diff = hv + rv - tv
                acc = acc + diff * diff
            scores_v[s, pl.ds(g * L, L)] = _neg_sqrt(acc + 1e-12)
            return ()

        lax.fori_loop(0, C // L, group, (), unroll=False)

        @pl.when(s < 2)
        def _():
            for k in range(N_NEG):
                pltpu.async_copy(
                    scores_v.at[s], pos_out.at[pl.ds(k * B + base + s * C, C)], sem_o
                )

        @pl.when(s >= 2)
        def _():
            k2 = s - 2
            dst = base + (k2 // 2) * B + (k2 % 2) * C
            pltpu.async_copy(scores_v.at[s], neg_out.at[pl.ds(dst, C)], sem_o)

        return ()

    lax.fori_loop(0, NSTAGE, stage_body, (), unroll=False)

    for s in range(2):
        for k in range(N_NEG):
            pltpu.make_async_copy(
                scores_v.at[s], pos_out.at[pl.ds(k * B + base + s * C, C)], sem_o
            ).wait()
    for s in range(2, NSTAGE):
        k2 = s - 2
        dst = base + (k2 // 2) * B + (k2 % 2) * C
        pltpu.make_async_copy(
            scores_v.at[s], neg_out.at[pl.ds(dst, C)], sem_o
        ).wait()


@jax.jit
def _run(heads, tails, relations, negative_head, negative_tails, entity_emb, relation_emb):
    mesh = plsc.VectorSubcoreMesh(
        core_axis_name="c", subcore_axis_name="s", num_cores=NC, num_subcores=NS
    )
    f = pl.kernel(
        _sc_kernel,
        out_type=(
            jax.ShapeDtypeStruct((NB,), jnp.float32),
            jax.ShapeDtypeStruct((NB,), jnp.float32),
        ),
        mesh=mesh,
        compiler_params=pltpu.CompilerParams(
            needs_layout_passes=False, use_tc_tiling_on_sc=False
        ),
        scratch_types=[
            pltpu.VMEM(((1 + N_NEG) * P // 128, 128), jnp.int32),
            pltpu.VMEM(((1 + N_NEG) * P // 128, 128), jnp.int32),
            pltpu.VMEM((P // 128, 128), jnp.int32),
            pltpu.VMEM((2, C, PAD), jnp.float32),
            pltpu.VMEM((2, C, PAD), jnp.float32),
            pltpu.VMEM((P, PAD), jnp.float32),
            pltpu.VMEM((NSTAGE, C), jnp.float32),
            pltpu.SemaphoreType.DMA,
            pltpu.SemaphoreType.DMA,
            pltpu.SemaphoreType.DMA,
            pltpu.SemaphoreType.DMA,
        ],
    )
    return f(heads, tails, relations, negative_head, negative_tails, entity_emb, relation_emb)


def kernel(heads, tails, relations, negative_head, negative_tails, entity_emb, relation_emb):
    return _run(
        heads.astype(jnp.int32),
        tails.astype(jnp.int32),
        relations.astype(jnp.int32),
        negative_head.astype(jnp.int32),
        negative_tails.astype(jnp.int32),
        entity_emb,
        relation_emb,
    )


# speedup vs baseline: 1.3232x; 1.2071x over previous
import jax
import jax.numpy as jnp
from jax import lax
from jax.experimental import pallas as pl
from jax.experimental.pallas import tpu as pltpu
from jax.experimental.pallas import tpu_sc as plsc

N_ENTITIES = 1000000
N_RELATIONS = 1000
D = 64
B = 16384
N_NEG = 4
NB = B * N_NEG

NC = 2
NS = 16
NW = NC * NS
L = 16

P = B // NW
C = 256
NSTAGE = (P + N_NEG * P) // C
PAD = 64
NSL = C // 128


def _neg_sqrt(x):
    i = plsc.bitcast(x, jnp.int32)
    i = 0x5F3759DF - lax.shift_right_arithmetic(i, 1)
    y = plsc.bitcast(i, jnp.float32)
    half = x * (-0.5)
    for _ in range(3):
        y = y * (1.5 + half * y * y)
    return -(x * y)


def _sc_kernel(
    heads,
    tails,
    relations,
    negative_head,
    negative_tails,
    entity_emb,
    relation_emb,
    pos_out,
    neg_out,
    idx_h,
    idx_t,
    idx_r,
    hbuf,
    tbuf,
    r_v,
    scores_v,
    sem_i,
    sem_r,
    sem_g,
    sem_o,
):
    wid = lax.axis_index("s") * NC + lax.axis_index("c")
    base = wid * P
    iota = lax.iota(jnp.int32, L)

    idx_copies = []
    for j in range(P // 128):
        idx_copies.append(
            pltpu.async_copy(
                relations.at[pl.ds(base + j * 128, 128)], idx_r.at[j], sem_i
            )
        )
        idx_copies.append(
            pltpu.async_copy(heads.at[pl.ds(base + j * 128, 128)], idx_h.at[j], sem_i)
        )
        idx_copies.append(
            pltpu.async_copy(tails.at[pl.ds(base + j * 128, 128)], idx_t.at[j], sem_i)
        )
    for k in range(N_NEG):
        for j in range(P // 128):
            src = k * B + base + j * 128
            row = P // 128 + k * (P // 128) + j
            idx_copies.append(
                pltpu.async_copy(
                    negative_head.at[pl.ds(src, 128)], idx_h.at[row], sem_i
                )
            )
            idx_copies.append(
                pltpu.async_copy(
                    negative_tails.at[pl.ds(src, 128)], idx_t.at[row], sem_i
                )
            )
    for c in idx_copies:
        c.wait()

    r_copies = [
        pltpu.async_copy(
            relation_emb.at[idx_r.at[j]],
            r_v.at[pl.ds(j * 128, 128)],
            sem_r,
        )
        for j in range(P // 128)
    ]
    for c in r_copies:
        c.wait()

    def fire(s):
        sp = s % 2
        for j in range(NSL):
            row = NSL * s + j
            pltpu.async_copy(
                entity_emb.at[idx_h.at[row]],
                hbuf.at[pl.ds(sp * C + j * 128, 128)],
                sem_g,
            )
            pltpu.async_copy(
                entity_emb.at[idx_t.at[row]],
                tbuf.at[pl.ds(sp * C + j * 128, 128)],
                sem_g,
            )

    def wait_stage(s):
        sp = s % 2
        for j in range(NSL):
            row = NSL * s + j
            pltpu.make_async_copy(
                entity_emb.at[idx_h.at[row]],
                hbuf.at[pl.ds(sp * C + j * 128, 128)],
                sem_g,
            ).wait()
            pltpu.make_async_copy(
                entity_emb.at[idx_t.at[row]],
                tbuf.at[pl.ds(sp * C + j * 128, 128)],
                sem_g,
            ).wait()

    fire(jnp.int32(0))

    def stage_body(s, _):
        wait_stage(s)

        @pl.when(s < NSTAGE - 1)
        def _():
            fire(s + 1)

        soff16 = jnp.full((L,), (s % 2) * C, jnp.int32)

        def group(g, _):
            rows = g * L + iota
            srows = soff16 + rows
            acc = jnp.zeros((L,), jnp.float32)
            for d in range(D):
                cols = (iota + d) & (D - 1)
                hv = plsc.load_gather(hbuf, [srows, cols])
                tv = plsc.load_gather(tbuf, [srows, cols])
                rv = plsc.load_gather(r_v, [srows, cols])
                diff = hv + rv - tv
                acc = acc + diff * diff
            scores_v[s, pl.ds(g * L, L)] = _neg_sqrt(acc + 1e-12)
            return ()

        lax.fori_loop(0, C // L, group, (), unroll=False)

        @pl.when(s < 2)
        def _():
            for k in range(N_NEG):
                pltpu.async_copy(
                    scores_v.at[s], pos_out.at[pl.ds(k * B + base + s * C, C)], sem_o
                )

        @pl.when(s >= 2)
        def _():
            k2 = s - 2
            dst = base + (k2 // 2) * B + (k2 % 2) * C
            pltpu.async_copy(scores_v.at[s], neg_out.at[pl.ds(dst, C)], sem_o)

        return ()

    lax.fori_loop(0, NSTAGE, stage_body, (), unroll=False)

    for s in range(2):
        for k in range(N_NEG):
            pltpu.make_async_copy(
                scores_v.at[s], pos_out.at[pl.ds(k * B + base + s * C, C)], sem_o
            ).wait()
    for s in range(2, NSTAGE):
        k2 = s - 2
        dst = base + (k2 // 2) * B + (k2 % 2) * C
        pltpu.make_async_copy(
            scores_v.at[s], neg_out.at[pl.ds(dst, C)], sem_o
        ).wait()


@jax.jit
def _run(heads, tails, relations, negative_head, negative_tails, entity_emb, relation_emb):
    mesh = plsc.VectorSubcoreMesh(
        core_axis_name="c", subcore_axis_name="s", num_cores=NC, num_subcores=NS
    )
    f = pl.kernel(
        _sc_kernel,
        out_type=(
            jax.ShapeDtypeStruct((NB,), jnp.float32),
            jax.ShapeDtypeStruct((NB,), jnp.float32),
        ),
        mesh=mesh,
        compiler_params=pltpu.CompilerParams(
            needs_layout_passes=False, use_tc_tiling_on_sc=False
        ),
        scratch_types=[
            pltpu.VMEM(((1 + N_NEG) * P // 128, 128), jnp.int32),
            pltpu.VMEM(((1 + N_NEG) * P // 128, 128), jnp.int32),
            pltpu.VMEM((P // 128, 128), jnp.int32),
            pltpu.VMEM((2 * C, PAD), jnp.float32),
            pltpu.VMEM((2 * C, PAD), jnp.float32),
            pltpu.VMEM((P, PAD), jnp.float32),
            pltpu.VMEM((NSTAGE, C), jnp.float32),
            pltpu.SemaphoreType.DMA,
            pltpu.SemaphoreType.DMA,
            pltpu.SemaphoreType.DMA,
            pltpu.SemaphoreType.DMA,
        ],
    )
    return f(heads, tails, relations, negative_head, negative_tails, entity_emb, relation_emb)


def kernel(heads, tails, relations, negative_head, negative_tails, entity_emb, relation_emb):
    return _run(
        heads.astype(jnp.int32),
        tails.astype(jnp.int32),
        relations.astype(jnp.int32),
        negative_head.astype(jnp.int32),
        negative_tails.astype(jnp.int32),
        entity_emb,
        relation_emb,
    )


# speedup vs baseline: 1.3618x; 1.0291x over previous
import jax
import jax.numpy as jnp
from jax import lax
from jax.experimental import pallas as pl
from jax.experimental.pallas import tpu as pltpu
from jax.experimental.pallas import tpu_sc as plsc

N_ENTITIES = 1000000
N_RELATIONS = 1000
D = 64
B = 16384
N_NEG = 4
NB = B * N_NEG

NC = 2
NS = 16
NW = NC * NS
L = 16

P = B // NW
C = 128
NSTAGE = (P + N_NEG * P) // C
NROW = (1 + N_NEG) * P // C


def _neg_sqrt(x):
    i = plsc.bitcast(x, jnp.int32)
    i = 0x5F3759DF - lax.shift_right_arithmetic(i, 1)
    y = plsc.bitcast(i, jnp.float32)
    half = x * (-0.5)
    for _ in range(3):
        y = y * (1.5 + half * y * y)
    return -(x * y)


def _sc_kernel(
    heads,
    tails,
    relations,
    negative_head,
    negative_tails,
    entity2,
    relation_emb,
    pos_out,
    neg_out,
    idx_h,
    idx_t,
    idx_r,
    idx2_h,
    idx2_t,
    hbuf,
    tbuf,
    r_v,
    scores_v,
    sem_i,
    sem_r,
    sem_g,
    sem_o,
):
    wid = lax.axis_index("s") * NC + lax.axis_index("c")
    base = wid * P
    iota = lax.iota(jnp.int32, L)

    idx_copies = []
    for j in range(P // C):
        idx_copies.append(
            pltpu.async_copy(relations.at[pl.ds(base + j * C, C)], idx_r.at[j], sem_i)
        )
        idx_copies.append(
            pltpu.async_copy(heads.at[pl.ds(base + j * C, C)], idx_h.at[j], sem_i)
        )
        idx_copies.append(
            pltpu.async_copy(tails.at[pl.ds(base + j * C, C)], idx_t.at[j], sem_i)
        )
    for k in range(N_NEG):
        for j in range(P // C):
            src = k * B + base + j * C
            row = P // C + k * (P // C) + j
            idx_copies.append(
                pltpu.async_copy(negative_head.at[pl.ds(src, C)], idx_h.at[row], sem_i)
            )
            idx_copies.append(
                pltpu.async_copy(negative_tails.at[pl.ds(src, C)], idx_t.at[row], sem_i)
            )
    for c in idx_copies:
        c.wait()

    r_copies = [
        pltpu.async_copy(
            relation_emb.at[idx_r.at[j]], r_v.at[pl.ds(j * C, C)], sem_r
        )
        for j in range(P // C)
    ]
    for c in r_copies:
        c.wait()

    def fire(s):
        sp = s % 2
        for v in range(C // L):
            sl = pl.ds(v * L, L)
            idx2_h[sp, sl] = lax.shift_right_logical(idx_h[s, sl], 1)
            idx2_t[sp, sl] = lax.shift_right_logical(idx_t[s, sl], 1)
        pltpu.async_copy(
            entity2.at[idx2_h.at[sp]], hbuf.at[pl.ds(sp * C, C)], sem_g
        )
        pltpu.async_copy(
            entity2.at[idx2_t.at[sp]], tbuf.at[pl.ds(sp * C, C)], sem_g
        )

    def wait_stage(s):
        sp = s % 2
        pltpu.make_async_copy(
            entity2.at[idx2_h.at[sp]], hbuf.at[pl.ds(sp * C, C)], sem_g
        ).wait()
        pltpu.make_async_copy(
            entity2.at[idx2_t.at[sp]], tbuf.at[pl.ds(sp * C, C)], sem_g
        ).wait()

    fire(jnp.int32(0))

    def stage_body(s, _):
        wait_stage(s)

        @pl.when(s < NSTAGE - 1)
        def _():
            fire(s + 1)

        soff16 = jnp.full((L,), (s % 2) * C, jnp.int32)
        roff16 = jnp.full((L,), (s % 4) * C, jnp.int32)

        def group(g, _):
            rows = g * L + iota
            srows = soff16 + rows
            rrows = roff16 + rows
            ph = (idx_h[s, pl.ds(g * L, L)] & 1) * D
            pt = (idx_t[s, pl.ds(g * L, L)] & 1) * D
            acc = jnp.zeros((L,), jnp.float32)
            for d in range(D):
                cols = (iota + d) & (D - 1)
                hv = plsc.load_gather(hbuf, [srows, ph + cols])
                tv = plsc.load_gather(tbuf, [srows, pt + cols])
                rv = plsc.load_gather(r_v, [rrows, cols])
                diff = hv + rv - tv
                acc = acc + diff * diff
            scores_v[s, pl.ds(g * L, L)] = _neg_sqrt(acc + 1e-12)
            return ()

        lax.fori_loop(0, C // L, group, (), unroll=False)

        @pl.when(s < P // C)
        def _():
            for k in range(N_NEG):
                pltpu.async_copy(
                    scores_v.at[s], pos_out.at[pl.ds(k * B + base + s * C, C)], sem_o
                )

        @pl.when(s >= P // C)
        def _():
            k2 = s - P // C
            dst = base + (k2 // (P // C)) * B + (k2 % (P // C)) * C
            pltpu.async_copy(scores_v.at[s], neg_out.at[pl.ds(dst, C)], sem_o)

        return ()

    lax.fori_loop(0, NSTAGE, stage_body, (), unroll=False)

    for s in range(P // C):
        for k in range(N_NEG):
            pltpu.make_async_copy(
                scores_v.at[s], pos_out.at[pl.ds(k * B + base + s * C, C)], sem_o
            ).wait()
    for s in range(P // C, NSTAGE):
        k2 = s - P // C
        dst = base + (k2 // (P // C)) * B + (k2 % (P // C)) * C
        pltpu.make_async_copy(
            scores_v.at[s], neg_out.at[pl.ds(dst, C)], sem_o
        ).wait()


@jax.jit
def _run(heads, tails, relations, negative_head, negative_tails, entity2, relation_emb):
    mesh = plsc.VectorSubcoreMesh(
        core_axis_name="c", subcore_axis_name="s", num_cores=NC, num_subcores=NS
    )
    f = pl.kernel(
        _sc_kernel,
        out_type=(
            jax.ShapeDtypeStruct((NB,), jnp.float32),
            jax.ShapeDtypeStruct((NB,), jnp.float32),
        ),
        mesh=mesh,
        compiler_params=pltpu.CompilerParams(
            needs_layout_passes=False, use_tc_tiling_on_sc=False
        ),
        scratch_types=[
            pltpu.VMEM((NROW, C), jnp.int32),
            pltpu.VMEM((NROW, C), jnp.int32),
            pltpu.VMEM((P // C, C), jnp.int32),
            pltpu.VMEM((2, C), jnp.int32),
            pltpu.VMEM((2, C), jnp.int32),
            pltpu.VMEM((2 * C, 2 * D), jnp.float32),
            pltpu.VMEM((2 * C, 2 * D), jnp.float32),
            pltpu.VMEM((P, D), jnp.float32),
            pltpu.VMEM((NSTAGE, C), jnp.float32),
            pltpu.SemaphoreType.DMA,
            pltpu.SemaphoreType.DMA,
            pltpu.SemaphoreType.DMA,
            pltpu.SemaphoreType.DMA,
        ],
    )
    return f(heads, tails, relations, negative_head, negative_tails, entity2, relation_emb)


def kernel(heads, tails, relations, negative_head, negative_tails, entity_emb, relation_emb):
    return _run(
        heads.astype(jnp.int32),
        tails.astype(jnp.int32),
        relations.astype(jnp.int32),
        negative_head.astype(jnp.int32),
        negative_tails.astype(jnp.int32),
        entity_emb.reshape(N_ENTITIES // 2, 2 * D),
        relation_emb,
    )
